# baseline (speedup 1.0000x reference)
import functools

import jax
import jax.numpy as jnp
from jax import lax
from jax.experimental import pallas as pl
from jax.experimental.pallas import tpu as pltpu
from jax.experimental.pallas import tpu_sc as plsc

D_EMB = 4096
N_SEQ = 8192
NC, NS = 2, 16
NW = NC * NS
ROWS_PER_W = N_SEQ // NW

_mesh = plsc.VectorSubcoreMesh(core_axis_name="c", subcore_axis_name="s")


@functools.partial(
    pl.kernel,
    mesh=_mesh,
    out_type=jax.ShapeDtypeStruct((N_SEQ, D_EMB), jnp.float32),
)
def _sc_copy(table_hbm, out_hbm):
    wid = lax.axis_index("s") * NC + lax.axis_index("c")
    base = wid * ROWS_PER_W
    pltpu.sync_copy(table_hbm.at[pl.ds(base, ROWS_PER_W)],
                    out_hbm.at[pl.ds(base, ROWS_PER_W)])


def kernel(x, table):
    del x
    return _sc_copy(table)


# speedup vs baseline: 35.2309x; 35.2309x over previous
import functools

import jax
import jax.numpy as jnp
from jax import lax
from jax.experimental import pallas as pl
from jax.experimental.pallas import tpu as pltpu
from jax.experimental.pallas import tpu_sc as plsc

D_EMB = 4096
N_SEQ = 8192
NC, NS = 2, 16
NW = NC * NS
ROWS_PER_W = N_SEQ // NW
CH = 8
NCH = ROWS_PER_W // CH

_mesh = plsc.VectorSubcoreMesh(core_axis_name="c", subcore_axis_name="s")


@functools.partial(
    pl.kernel,
    mesh=_mesh,
    out_type=jax.ShapeDtypeStruct((N_SEQ, D_EMB), jnp.float32),
    scratch_types=[
        pltpu.VMEM((CH, D_EMB), jnp.float32),
        pltpu.VMEM((CH, D_EMB), jnp.float32),
        pltpu.SemaphoreType.DMA,
        pltpu.SemaphoreType.DMA,
        pltpu.SemaphoreType.DMA,
        pltpu.SemaphoreType.DMA,
    ],
)
def _sc_copy(table_hbm, out_hbm, buf0, buf1, sr0, sr1, sw0, sw1):
    wid = lax.axis_index("s") * NC + lax.axis_index("c")
    base = wid * ROWS_PER_W
    bufs = (buf0, buf1)
    srs = (sr0, sr1)
    sws = (sw0, sw1)
    writes = [None, None]
    for i in range(NCH):
        b = i % 2
        r0 = base + i * CH
        if writes[b] is not None:
            writes[b].wait()
        pltpu.async_copy(table_hbm.at[pl.ds(r0, CH)], bufs[b], srs[b]).wait()
        writes[b] = pltpu.async_copy(bufs[b], out_hbm.at[pl.ds(r0, CH)], sws[b])
    writes[0].wait()
    writes[1].wait()


def kernel(x, table):
    del x
    return _sc_copy(table)
